# baseline (speedup 1.0000x reference)
import functools

import jax
import jax.numpy as jnp
from jax.experimental import pallas as pl
from jax.experimental.pallas import tpu as pltpu

_NUM_EMB = 1024
_DIM = 64
_PDIM = 32
_N = 32768
_R = 256


def _vq_kernel(z_ref, emb_ref, w_ref, b_ref, q_ref, oh_ref, embp_ref, embn_ref):
    i = pl.program_id(0)

    @pl.when(i == 0)
    def _():
        emb_p = jax.lax.dot_general(
            emb_ref[:], w_ref[:], (((1,), (1,)), ((), ())),
            preferred_element_type=jnp.float32) + b_ref[:]
        embp_ref[:] = emb_p
        embn_ref[:] = jnp.sum(emb_p * emb_p, axis=1)[None, :]

    z_p = jax.lax.dot_general(
        z_ref[:], w_ref[:], (((1,), (1,)), ((), ())),
        preferred_element_type=jnp.float32) + b_ref[:]
    rowsq = jnp.sum(z_p * z_p, axis=1, keepdims=True)
    cross = jax.lax.dot_general(
        z_p, embp_ref[:], (((1,), (1,)), ((), ())),
        preferred_element_type=jnp.float32)
    dist = (rowsq + embn_ref[:]) - 2.0 * cross
    closest = jnp.argmin(dist, axis=1).astype(jnp.int32)
    oh = (jax.lax.broadcasted_iota(jnp.int32, (_R, _NUM_EMB), 1)
          == closest[:, None]).astype(jnp.float32)
    oh_ref[:] = oh
    q_ref[:] = jnp.dot(oh, emb_ref[:], preferred_element_type=jnp.float32)


@functools.partial(jax.jit)
def kernel(z, embeddings, W, b):
    b2 = b.reshape(1, _PDIM)
    quantized, one_hot = pl.pallas_call(
        _vq_kernel,
        grid=(_N // _R,),
        in_specs=[
            pl.BlockSpec((_R, _DIM), lambda i: (i, 0)),
            pl.BlockSpec((_NUM_EMB, _DIM), lambda i: (0, 0)),
            pl.BlockSpec((_PDIM, _DIM), lambda i: (0, 0)),
            pl.BlockSpec((1, _PDIM), lambda i: (0, 0)),
        ],
        out_specs=[
            pl.BlockSpec((_R, _DIM), lambda i: (i, 0)),
            pl.BlockSpec((_R, _NUM_EMB), lambda i: (i, 0)),
        ],
        out_shape=[
            jax.ShapeDtypeStruct((_N, _DIM), jnp.float32),
            jax.ShapeDtypeStruct((_N, _NUM_EMB), jnp.float32),
        ],
        scratch_shapes=[
            pltpu.VMEM((_NUM_EMB, _PDIM), jnp.float32),
            pltpu.VMEM((1, _NUM_EMB), jnp.float32),
        ],
    )(z, embeddings, W, b2)
    return (quantized, one_hot)
